# Initial kernel scaffold; baseline (speedup 1.0000x reference)
import jax
import jax.numpy as jnp
from jax.experimental import pallas as pl


def kernel(x, edge_index, W1l, b1l, W1r, gamma, beta, W2l, b2l, W2r, Wc, bc):
    raise NotImplementedError("write your pallas kernel here")


# speedup vs baseline: 6.9715x; 6.9715x over previous
import functools

import jax
import jax.numpy as jnp
from jax import lax
from jax.experimental import pallas as pl
from jax.experimental.pallas import tpu as pltpu
from jax.experimental.pallas import tpu_sc as plsc

N_NODES = 10000
N_EDGES = 320000
D_IN = 128
D_HID = 64
D_OUT = 16
N_CLS = 2

D1 = D_HID + 16
NC = 2
NS = 16
NW = NC * NS
CHUNK = 128
EPW = 10240
NCHUNKS = EPW // CHUNK
E_PAD = NW * EPW
N_PAD = 10016
STRIPE = N_PAD // NS


def _sc_segsum(d_feat):
    mesh = plsc.VectorSubcoreMesh(core_axis_name="c", subcore_axis_name="s")

    @functools.partial(
        pl.kernel,
        out_type=jax.ShapeDtypeStruct((NC, N_PAD, d_feat), jnp.float32),
        mesh=mesh,
        scratch_types=[
            pltpu.VMEM((NCHUNKS, CHUNK), jnp.int32),
            pltpu.VMEM((NCHUNKS, CHUNK), jnp.int32),
            pltpu.VMEM((CHUNK, d_feat), jnp.float32),
            pltpu.VMEM((STRIPE, d_feat), jnp.float32),
            pltpu.VMEM_SHARED((N_PAD, d_feat), jnp.float32),
            pltpu.SemaphoreType.DMA,
        ],
        compiler_params=pltpu.CompilerParams(use_tc_tiling_on_sc=False),
    )
    def seg_kernel(tbl_h, src_h, dst_h, agg_h,
                   src_v, dst_v, rows_v, zero_v, acc_sh, sem):
        c = lax.axis_index("c")
        s = lax.axis_index("s")

        pltpu.sync_copy(src_h.at[c, s], src_v)
        pltpu.sync_copy(dst_h.at[c, s], dst_v)

        zvec = jnp.zeros((16,), jnp.float32)

        def zrow(i, carry):
            def zcol(j, cc):
                zero_v[i, pl.ds(j * 16, 16)] = zvec
                return cc
            return lax.fori_loop(0, d_feat // 16, zcol, carry)
        lax.fori_loop(0, STRIPE, zrow, 0)
        pltpu.sync_copy(zero_v, acc_sh.at[pl.ds(s * STRIPE, STRIPE)])

        plsc.subcore_barrier()

        def chunk_body(j, carry):
            pltpu.async_copy(tbl_h.at[src_v.at[j]], rows_v, sem).wait()
            pltpu.sync_copy(rows_v, acc_sh.at[dst_v.at[j]], add=True)
            return carry
        lax.fori_loop(0, NCHUNKS, chunk_body, 0)

        plsc.subcore_barrier()

        pltpu.sync_copy(acc_sh.at[pl.ds(s * STRIPE, STRIPE)], zero_v)
        pltpu.sync_copy(zero_v, agg_h.at[c, pl.ds(s * STRIPE, STRIPE)])

    return seg_kernel


_seg1 = _sc_segsum(D1)
_seg2 = _sc_segsum(D_OUT)


def _tc1_body(x_ref, w1l_ref, w1r_ref, p1_ref, q1_ref):
    x = x_ref[...]
    p1_ref[:, :D_HID] = jnp.dot(x, w1l_ref[...],
                                preferred_element_type=jnp.float32)
    p1_ref[:, D_HID:] = jnp.ones((N_NODES, 16), jnp.float32)
    q1_ref[...] = jnp.dot(x, w1r_ref[...], preferred_element_type=jnp.float32)


def _tc2_body(agg_ref, q1_ref, b1l_ref, gamma_ref, beta_ref,
              w2l_ref, w2r_ref, p2_ref, q2_ref, rdeg_ref):
    agg = agg_ref[0, :N_NODES, :D_HID] + agg_ref[1, :N_NODES, :D_HID]
    cnt = agg_ref[0, :N_NODES, D_HID] + agg_ref[1, :N_NODES, D_HID]
    rdeg = (1.0 / jnp.maximum(cnt, 1.0))[:, None]
    h = agg * rdeg + b1l_ref[...][None, :] + q1_ref[...]
    mu = jnp.mean(h, axis=0)
    d = h - mu[None, :]
    var = jnp.mean(d * d, axis=0)
    hn = d * (gamma_ref[...] / jnp.sqrt(var + 1e-5))[None, :] + beta_ref[...][None, :]
    hn = jnp.maximum(hn, 0.0)
    p2_ref[...] = jnp.dot(hn, w2l_ref[...], preferred_element_type=jnp.float32)
    q2_ref[...] = jnp.dot(hn, w2r_ref[...], preferred_element_type=jnp.float32)
    rdeg_ref[...] = rdeg


def _tc3_body(agg2_ref, rdeg_ref, q2_ref, b2l_ref, wc_ref, bc_ref,
              logits_ref, emb_ref):
    agg2 = agg2_ref[0, :N_NODES, :] + agg2_ref[1, :N_NODES, :]
    emb = agg2 * rdeg_ref[...] + b2l_ref[...][None, :] + q2_ref[...]
    logits_ref[...] = (
        jnp.dot(emb, wc_ref[...], preferred_element_type=jnp.float32)
        + bc_ref[...][None, :])
    emb_ref[...] = emb


@jax.jit
def kernel(x, edge_index, W1l, b1l, W1r, gamma, beta, W2l, b2l, W2r, Wc, bc):
    src = edge_index[0]
    dst = edge_index[1]
    pad = E_PAD - N_EDGES
    src_p = jnp.concatenate(
        [src, jnp.zeros((pad,), jnp.int32)]).reshape(NC, NS, NCHUNKS, CHUNK)
    dst_p = jnp.concatenate(
        [dst, jnp.full((pad,), N_NODES, jnp.int32)]).reshape(NC, NS, NCHUNKS, CHUNK)

    p1, q1 = pl.pallas_call(
        _tc1_body,
        out_shape=[
            jax.ShapeDtypeStruct((N_NODES, D1), jnp.float32),
            jax.ShapeDtypeStruct((N_NODES, D_HID), jnp.float32),
        ],
    )(x, W1l, W1r)

    agg1 = _seg1(p1, src_p, dst_p)

    p2, q2, rdeg = pl.pallas_call(
        _tc2_body,
        out_shape=[
            jax.ShapeDtypeStruct((N_NODES, D_OUT), jnp.float32),
            jax.ShapeDtypeStruct((N_NODES, D_OUT), jnp.float32),
            jax.ShapeDtypeStruct((N_NODES, 1), jnp.float32),
        ],
    )(agg1, q1, b1l, gamma, beta, W2l, W2r)

    agg2 = _seg2(p2, src_p, dst_p)

    logits, emb = pl.pallas_call(
        _tc3_body,
        out_shape=[
            jax.ShapeDtypeStruct((N_NODES, N_CLS), jnp.float32),
            jax.ShapeDtypeStruct((N_NODES, D_OUT), jnp.float32),
        ],
    )(agg2, rdeg, q2, b2l, Wc, bc)

    return (logits, emb)
